# baseline (speedup 1.0000x reference)
import functools

import jax
import jax.numpy as jnp
from jax import lax
from jax.experimental import pallas as pl
from jax.experimental.pallas import tpu as pltpu
from jax.experimental.pallas import tpu_sc as plsc

_N = 10000
_E = 160000
_D = 256
_DH = 128
_EA = 16
_H2 = 128

_NC = 2
_NS = 16
_NW = _NC * _NS
_CHUNK = 128
_CPW0 = 56
_CPW1 = 24
_FAST_CORE = 1
_SLOTS = 3
_EP = _NS * (_CPW0 + _CPW1) * _CHUNK


def _pack_bf16_pair(p):
    p16 = p.astype(jnp.bfloat16).astype(jnp.float32)
    lo = lax.bitcast_convert_type(p16[:, :_DH], jnp.uint32) >> 16
    hi = lax.bitcast_convert_type(p16[:, _DH:], jnp.uint32)
    return lo | hi


def _unpack_bf16_pair(g):
    lo = lax.bitcast_convert_type(g << 16, jnp.float32)
    hi = lax.bitcast_convert_type(g & jnp.uint32(0xFFFF0000), jnp.float32)
    return lo, hi


def _node_proj_body(x_ref, wa_ref, wb_ref, pa_ref, pb_ref):
    x = x_ref[...]
    pa_ref[...] = _pack_bf16_pair(
        jnp.dot(x, wa_ref[...], preferred_element_type=jnp.float32))
    pb_ref[...] = _pack_bf16_pair(
        jnp.dot(x, wb_ref[...], preferred_element_type=jnp.float32))


def _node_proj(nodes, wa, wb):
    blk = 2000
    return pl.pallas_call(
        _node_proj_body,
        grid=(_N // blk,),
        in_specs=[
            pl.BlockSpec((blk, _D), lambda i: (i, 0)),
            pl.BlockSpec((_D, _D), lambda i: (0, 0)),
            pl.BlockSpec((_D, _D), lambda i: (0, 0)),
        ],
        out_specs=[
            pl.BlockSpec((blk, _DH), lambda i: (i, 0)),
            pl.BlockSpec((blk, _DH), lambda i: (i, 0)),
        ],
        out_shape=[
            jax.ShapeDtypeStruct((_N, _DH), jnp.uint32),
            jax.ShapeDtypeStruct((_N, _DH), jnp.uint32),
        ],
    )(nodes, wa, wb)


def _sc_gather(pa, pb, ridx2, cidx2):
    mesh = plsc.VectorSubcoreMesh(core_axis_name="c", subcore_axis_name="s",
                                  num_cores=_NC, num_subcores=_NS)

    nslots = _SLOTS

    @functools.partial(
        pl.kernel,
        out_type=(jax.ShapeDtypeStruct((_EP, _DH), jnp.uint32),
                  jax.ShapeDtypeStruct((_EP, _DH), jnp.uint32)),
        mesh=mesh,
        scratch_types=(
            [pltpu.VMEM((_CPW0, _CHUNK), jnp.int32)] * 2
            + [pltpu.VMEM((_CHUNK, _DH), jnp.uint32)] * (2 * nslots)
            + [pltpu.SemaphoreType.DMA] * (4 * nslots)
        ),
    )
    def body(pa_hbm, pb_hbm, ridx_hbm, cidx_hbm, ga_hbm, gb_hbm, *scr):
        ridx_v, cidx_v = scr[0], scr[1]
        bufa = scr[2:2 + nslots]
        bufb = scr[2 + nslots:2 + 2 * nslots]
        sems = scr[2 + 2 * nslots:]
        sa, sb = sems[:nslots], sems[nslots:2 * nslots]
        wsa, wsb = sems[2 * nslots:3 * nslots], sems[3 * nslots:]

        cid = lax.axis_index("c")
        sid = lax.axis_index("s")

        def pipe(cpw, cbase):
            pltpu.sync_copy(ridx_hbm.at[pl.ds(cbase, cpw)],
                            ridx_v.at[pl.ds(0, cpw)])
            pltpu.sync_copy(cidx_hbm.at[pl.ds(cbase, cpw)],
                            cidx_v.at[pl.ds(0, cpw)])
            ebase = cbase * _CHUNK

            def gath(j):
                t = j % nslots
                return (pltpu.async_copy(pa_hbm.at[ridx_v.at[j]],
                                         bufa[t], sa[t]),
                        pltpu.async_copy(pb_hbm.at[cidx_v.at[j]],
                                         bufb[t], sb[t]))

            h = [None] * nslots
            wh = [None] * nslots
            for j in range(min(nslots - 1, cpw)):
                h[j % nslots] = gath(j)
            for i in range(cpw):
                s = i % nslots
                j = i + nslots - 1
                if j < cpw:
                    t = j % nslots
                    if wh[t] is not None:
                        wh[t][0].wait()
                        wh[t][1].wait()
                    h[t] = gath(j)
                h[s][0].wait()
                h[s][1].wait()
                off = ebase + i * _CHUNK
                wh[s] = (
                    pltpu.async_copy(bufa[s],
                                     ga_hbm.at[pl.ds(off, _CHUNK)], wsa[s]),
                    pltpu.async_copy(bufb[s],
                                     gb_hbm.at[pl.ds(off, _CHUNK)], wsb[s]))
            for s in range(nslots):
                if wh[s] is not None:
                    wh[s][0].wait()
                    wh[s][1].wait()

        lax.cond(cid == _FAST_CORE,
                 lambda: pipe(_CPW0, sid * _CPW0),
                 lambda: pipe(_CPW1, _NS * _CPW0 + sid * _CPW1))

    return body(pa, pb, ridx2, cidx2)


def _mlp_body(ga_ref, gb_ref, ea_ref, w1c_ref, b1_ref, w2_ref, b2_ref, w3t_ref,
              b3_ref, o_ref):
    ga_lo, ga_hi = _unpack_bf16_pair(ga_ref[...])
    gb_lo, gb_hi = _unpack_bf16_pair(gb_ref[...])
    ec = (lax.dot_general(ea_ref[...], w1c_ref[...],
                          (((0,), (0,)), ((), ())),
                          preferred_element_type=jnp.float32)
          + b1_ref[...])
    h1_lo = jnp.maximum(ga_lo + gb_lo + ec[:, :_DH], 0.0)
    h1_hi = jnp.maximum(ga_hi + gb_hi + ec[:, _DH:], 0.0)
    h2 = (jnp.dot(h1_lo, w2_ref[:_DH, :], preferred_element_type=jnp.float32)
          + jnp.dot(h1_hi, w2_ref[_DH:, :], preferred_element_type=jnp.float32)
          + b2_ref[...])
    h2 = jnp.maximum(h2, 0.0)
    z = jnp.sum(h2 * w3t_ref[...], axis=1) + b3_ref[0, 0]
    o_ref[...] = jax.nn.sigmoid(z).reshape(o_ref.shape)


def _mlp(ga, gb, ea_t, w1c, b1, w2, b2, w3t, b3):
    blk = 6400
    return pl.pallas_call(
        _mlp_body,
        grid=(_E // blk,),
        in_specs=[
            pl.BlockSpec((blk, _DH), lambda i: (i, 0)),
            pl.BlockSpec((blk, _DH), lambda i: (i, 0)),
            pl.BlockSpec((_EA, blk), lambda i: (0, i)),
            pl.BlockSpec((_EA, _D), lambda i: (0, 0)),
            pl.BlockSpec((1, _D), lambda i: (0, 0)),
            pl.BlockSpec((_D, _H2), lambda i: (0, 0)),
            pl.BlockSpec((1, _H2), lambda i: (0, 0)),
            pl.BlockSpec((1, _H2), lambda i: (0, 0)),
            pl.BlockSpec((1, 1), lambda i: (0, 0)),
        ],
        out_specs=pl.BlockSpec((1, blk // 128, 128), lambda i: (i, 0, 0)),
        out_shape=jax.ShapeDtypeStruct((_E // blk, blk // 128, 128),
                                       jnp.float32),
    )(ga, gb, ea_t, w1c, b1, w2, b2, w3t, b3)


def kernel(node_features, edge_index, edge_attr, W1, b1, W2, b2, W3, b3):
    row = edge_index[0].astype(jnp.int32)
    col = edge_index[1].astype(jnp.int32)
    pad = _EP - _E
    ridx2 = jnp.pad(row, (0, pad)).reshape(_EP // _CHUNK, _CHUNK)
    cidx2 = jnp.pad(col, (0, pad)).reshape(_EP // _CHUNK, _CHUNK)

    wa = W1[:_D]
    wb = W1[_D:2 * _D]
    w1c = W1[2 * _D:]

    pa, pb = _node_proj(node_features, wa, wb)
    ga, gb = _sc_gather(pa, pb, ridx2, cidx2)

    out = _mlp(ga, gb, edge_attr.T, w1c,
               b1.reshape(1, _D), W2, b2.reshape(1, _H2),
               W3.reshape(1, _H2), b3.reshape(1, 1))
    return out.reshape(_E, 1)


# speedup vs baseline: 1.0169x; 1.0169x over previous
import functools

import jax
import jax.numpy as jnp
from jax import lax
from jax.experimental import pallas as pl
from jax.experimental.pallas import tpu as pltpu
from jax.experimental.pallas import tpu_sc as plsc

_N = 10000
_E = 160000
_D = 256
_DH = 128
_EA = 16
_H2 = 128

_NC = 2
_NS = 16
_NW = _NC * _NS
_CHUNK = 128
_CPW0 = 56
_CPW1 = 24
_FAST_CORE = 0
_SLOTS = 3
_EP = _NS * (_CPW0 + _CPW1) * _CHUNK


def _pack_bf16_pair(p):
    p16 = p.astype(jnp.bfloat16).astype(jnp.float32)
    lo = lax.bitcast_convert_type(p16[:, :_DH], jnp.uint32) >> 16
    hi = lax.bitcast_convert_type(p16[:, _DH:], jnp.uint32)
    return lo | hi


def _unpack_bf16_pair(g):
    lo = lax.bitcast_convert_type(g << 16, jnp.float32)
    hi = lax.bitcast_convert_type(g & jnp.uint32(0xFFFF0000), jnp.float32)
    return lo, hi


def _node_proj_body(x_ref, wa_ref, wb_ref, pa_ref, pb_ref):
    x = x_ref[...]
    pa_ref[...] = _pack_bf16_pair(
        jnp.dot(x, wa_ref[...], preferred_element_type=jnp.float32))
    pb_ref[...] = _pack_bf16_pair(
        jnp.dot(x, wb_ref[...], preferred_element_type=jnp.float32))


def _node_proj(nodes, wa, wb):
    blk = 2000
    return pl.pallas_call(
        _node_proj_body,
        grid=(_N // blk,),
        in_specs=[
            pl.BlockSpec((blk, _D), lambda i: (i, 0)),
            pl.BlockSpec((_D, _D), lambda i: (0, 0)),
            pl.BlockSpec((_D, _D), lambda i: (0, 0)),
        ],
        out_specs=[
            pl.BlockSpec((blk, _DH), lambda i: (i, 0)),
            pl.BlockSpec((blk, _DH), lambda i: (i, 0)),
        ],
        out_shape=[
            jax.ShapeDtypeStruct((_N, _DH), jnp.uint32),
            jax.ShapeDtypeStruct((_N, _DH), jnp.uint32),
        ],
    )(nodes, wa, wb)


def _sc_gather(pa, pb, ridx2, cidx2):
    mesh = plsc.VectorSubcoreMesh(core_axis_name="c", subcore_axis_name="s",
                                  num_cores=_NC, num_subcores=_NS)

    nslots = _SLOTS

    @functools.partial(
        pl.kernel,
        out_type=(jax.ShapeDtypeStruct((_EP, _DH), jnp.uint32),
                  jax.ShapeDtypeStruct((_EP, _DH), jnp.uint32)),
        mesh=mesh,
        scratch_types=(
            [pltpu.VMEM((_CPW0, _CHUNK), jnp.int32)] * 2
            + [pltpu.VMEM((_CHUNK, _DH), jnp.uint32)] * (2 * nslots)
            + [pltpu.SemaphoreType.DMA] * (4 * nslots)
        ),
    )
    def body(pa_hbm, pb_hbm, ridx_hbm, cidx_hbm, ga_hbm, gb_hbm, *scr):
        ridx_v, cidx_v = scr[0], scr[1]
        bufa = scr[2:2 + nslots]
        bufb = scr[2 + nslots:2 + 2 * nslots]
        sems = scr[2 + 2 * nslots:]
        sa, sb = sems[:nslots], sems[nslots:2 * nslots]
        wsa, wsb = sems[2 * nslots:3 * nslots], sems[3 * nslots:]

        cid = lax.axis_index("c")
        sid = lax.axis_index("s")

        def pipe(cpw, cbase):
            pltpu.sync_copy(ridx_hbm.at[pl.ds(cbase, cpw)],
                            ridx_v.at[pl.ds(0, cpw)])
            pltpu.sync_copy(cidx_hbm.at[pl.ds(cbase, cpw)],
                            cidx_v.at[pl.ds(0, cpw)])
            ebase = cbase * _CHUNK

            def gath(j):
                t = j % nslots
                return (pltpu.async_copy(pa_hbm.at[ridx_v.at[j]],
                                         bufa[t], sa[t]),
                        pltpu.async_copy(pb_hbm.at[cidx_v.at[j]],
                                         bufb[t], sb[t]))

            h = [None] * nslots
            wh = [None] * nslots
            for j in range(min(nslots - 1, cpw)):
                h[j % nslots] = gath(j)
            for i in range(cpw):
                s = i % nslots
                j = i + nslots - 1
                if j < cpw:
                    t = j % nslots
                    if wh[t] is not None:
                        wh[t][0].wait()
                        wh[t][1].wait()
                    h[t] = gath(j)
                h[s][0].wait()
                h[s][1].wait()
                off = ebase + i * _CHUNK
                wh[s] = (
                    pltpu.async_copy(bufa[s],
                                     ga_hbm.at[pl.ds(off, _CHUNK)], wsa[s]),
                    pltpu.async_copy(bufb[s],
                                     gb_hbm.at[pl.ds(off, _CHUNK)], wsb[s]))
            for s in range(nslots):
                if wh[s] is not None:
                    wh[s][0].wait()
                    wh[s][1].wait()

        lax.cond(cid == _FAST_CORE,
                 lambda: pipe(_CPW0, sid * _CPW0),
                 lambda: pipe(_CPW1, _NS * _CPW0 + sid * _CPW1))

    return body(pa, pb, ridx2, cidx2)


def _mlp_body(ga_ref, gb_ref, ea_ref, w1c_ref, b1_ref, w2_ref, b2_ref, w3t_ref,
              b3_ref, o_ref):
    ga_lo, ga_hi = _unpack_bf16_pair(ga_ref[...])
    gb_lo, gb_hi = _unpack_bf16_pair(gb_ref[...])
    ec = (lax.dot_general(ea_ref[...], w1c_ref[...],
                          (((0,), (0,)), ((), ())),
                          preferred_element_type=jnp.float32)
          + b1_ref[...])
    h1_lo = jnp.maximum(ga_lo + gb_lo + ec[:, :_DH], 0.0)
    h1_hi = jnp.maximum(ga_hi + gb_hi + ec[:, _DH:], 0.0)
    h2 = (jnp.dot(h1_lo, w2_ref[:_DH, :], preferred_element_type=jnp.float32)
          + jnp.dot(h1_hi, w2_ref[_DH:, :], preferred_element_type=jnp.float32)
          + b2_ref[...])
    h2 = jnp.maximum(h2, 0.0)
    z = jnp.sum(h2 * w3t_ref[...], axis=1) + b3_ref[0, 0]
    o_ref[...] = jax.nn.sigmoid(z).reshape(o_ref.shape)


def _mlp(ga, gb, ea_t, w1c, b1, w2, b2, w3t, b3):
    blk = 6400
    return pl.pallas_call(
        _mlp_body,
        grid=(_E // blk,),
        in_specs=[
            pl.BlockSpec((blk, _DH), lambda i: (i, 0)),
            pl.BlockSpec((blk, _DH), lambda i: (i, 0)),
            pl.BlockSpec((_EA, blk), lambda i: (0, i)),
            pl.BlockSpec((_EA, _D), lambda i: (0, 0)),
            pl.BlockSpec((1, _D), lambda i: (0, 0)),
            pl.BlockSpec((_D, _H2), lambda i: (0, 0)),
            pl.BlockSpec((1, _H2), lambda i: (0, 0)),
            pl.BlockSpec((1, _H2), lambda i: (0, 0)),
            pl.BlockSpec((1, 1), lambda i: (0, 0)),
        ],
        out_specs=pl.BlockSpec((1, blk // 128, 128), lambda i: (i, 0, 0)),
        out_shape=jax.ShapeDtypeStruct((_E // blk, blk // 128, 128),
                                       jnp.float32),
    )(ga, gb, ea_t, w1c, b1, w2, b2, w3t, b3)


def kernel(node_features, edge_index, edge_attr, W1, b1, W2, b2, W3, b3):
    row = edge_index[0].astype(jnp.int32)
    col = edge_index[1].astype(jnp.int32)
    pad = _EP - _E
    ridx2 = jnp.pad(row, (0, pad)).reshape(_EP // _CHUNK, _CHUNK)
    cidx2 = jnp.pad(col, (0, pad)).reshape(_EP // _CHUNK, _CHUNK)

    wa = W1[:_D]
    wb = W1[_D:2 * _D]
    w1c = W1[2 * _D:]

    pa, pb = _node_proj(node_features, wa, wb)
    ga, gb = _sc_gather(pa, pb, ridx2, cidx2)

    out = _mlp(ga, gb, edge_attr.T, w1c,
               b1.reshape(1, _D), W2, b2.reshape(1, _H2),
               W3.reshape(1, _H2), b3.reshape(1, 1))
    return out.reshape(_E, 1)
